# Initial kernel scaffold; baseline (speedup 1.0000x reference)
import jax
import jax.numpy as jnp
from jax.experimental import pallas as pl


def kernel(x, edge_index, W1, b1, W2, b2):
    raise NotImplementedError("write your pallas kernel here")


# speedup vs baseline: 5.1226x; 5.1226x over previous
import functools

import jax
import jax.numpy as jnp
from jax import lax
from jax.experimental import pallas as pl
from jax.experimental.pallas import tpu as pltpu
from jax.experimental.pallas import tpu_sc as plsc

NC = 2
NS = 16
NW = NC * NS


def _make_seg_sum(n, e, d, ch):
  et = e // NW
  nchunks = et // ch
  rpt = (n // NS + 7) // 8 * 8
  npad = rpt * NS
  mesh = plsc.VectorSubcoreMesh(core_axis_name="c", subcore_axis_name="s")

  @functools.partial(
      pl.kernel,
      mesh=mesh,
      compiler_params=pltpu.CompilerParams(use_tc_tiling_on_sc=False),
      out_type=jax.ShapeDtypeStruct((NC, npad, d), jnp.float32),
      scratch_types=[
          pltpu.VMEM((ch,), jnp.int32),
          pltpu.VMEM((1, ch), jnp.int32),
          pltpu.VMEM((ch, d), jnp.float32),
          pltpu.VMEM_SHARED((npad, d), jnp.float32),
          pltpu.SemaphoreType.DMA,
      ],
  )
  def seg(tab_hbm, src_hbm, dst_hbm, zrows_hbm, out_hbm,
          sidx_v, didx_v, rows_v, table_s, sem):
    c = lax.axis_index("c")
    s = lax.axis_index("s")
    wid = s * NC + c

    pltpu.sync_copy(zrows_hbm, table_s.at[pl.ds(s * rpt, rpt)])
    plsc.subcore_barrier()

    ebase = wid * et

    def step(j, carry):
      eb = ebase + j * ch
      pltpu.sync_copy(src_hbm.at[pl.ds(eb, ch)], sidx_v)
      pltpu.sync_copy(dst_hbm.at[pl.ds(eb, ch)], didx_v.at[0])
      pltpu.async_copy(tab_hbm.at[sidx_v], rows_v, sem).wait()
      pltpu.sync_copy(rows_v, table_s.at[didx_v.at[0]], add=True)
      return carry

    lax.fori_loop(0, nchunks, step, 0)
    plsc.subcore_barrier()

    pltpu.sync_copy(table_s.at[pl.ds(s * rpt, rpt)],
                    out_hbm.at[c, pl.ds(s * rpt, rpt)])

  return seg


def _layer1_body(agg_ref, x_ref, w1_ref, b1_ref, w2_ref, p_ref, invd_ref):
  agg = agg_ref[0] + agg_ref[1]
  din = x_ref.shape[1]
  aggx = agg[:, :din] + x_ref[...]
  deg = jnp.sum(agg[:, din:din + 16], axis=1, keepdims=True)
  inv = 1.0 / (deg + 1.0)
  h = aggx * inv
  h1 = jnp.maximum(
      jnp.dot(h, w1_ref[...], preferred_element_type=jnp.float32)
      + b1_ref[...], 0.0)
  p_ref[...] = jnp.dot(h1, w2_ref[...], preferred_element_type=jnp.float32)
  invd_ref[...] = jnp.broadcast_to(inv, invd_ref.shape)


def _layer2_body(agg_ref, p_ref, invd_ref, b2_ref, z_ref):
  z_ref[...] = ((agg_ref[0] + agg_ref[1] + p_ref[...]) * invd_ref[...]
                + b2_ref[...])


def _decoder_body(zr_ref, zc_ref, out_ref):
  out_ref[...] = lax.dot_general(
      zr_ref[...], zc_ref[...], (((1,), (1,)), ((), ())),
      preferred_element_type=jnp.float32)


def kernel(x, edge_index, W1, b1, W2, b2):
  n, din = x.shape
  e = edge_index.shape[1]
  h1_dim = W1.shape[1]
  h2 = W2.shape[1]
  dp = din + 16

  src = edge_index[0]
  dst = edge_index[1]

  xa = jnp.concatenate(
      [x, jnp.ones((n, 1), jnp.float32), jnp.zeros((n, 15), jnp.float32)],
      axis=1)

  rpt = (n // NS + 7) // 8 * 8
  z1 = jnp.zeros((rpt, dp), jnp.float32)
  z2 = jnp.zeros((rpt, h2), jnp.float32)

  seg1 = _make_seg_sum(n, e, dp, 80)
  seg2 = _make_seg_sum(n, e, h2, 80)

  agg1 = seg1(xa, src, dst, z1)

  rblk = 2000
  grid1 = n // rblk
  p, invd = pl.pallas_call(
      _layer1_body,
      grid=(grid1,),
      in_specs=[
          pl.BlockSpec((NC, rblk, dp), lambda i: (0, i, 0)),
          pl.BlockSpec((rblk, din), lambda i: (i, 0)),
          pl.BlockSpec((din, h1_dim), lambda i: (0, 0)),
          pl.BlockSpec((1, h1_dim), lambda i: (0, 0)),
          pl.BlockSpec((h1_dim, h2), lambda i: (0, 0)),
      ],
      out_specs=[
          pl.BlockSpec((rblk, h2), lambda i: (i, 0)),
          pl.BlockSpec((rblk, h2), lambda i: (i, 0)),
      ],
      out_shape=[
          jax.ShapeDtypeStruct((n, h2), jnp.float32),
          jax.ShapeDtypeStruct((n, h2), jnp.float32),
      ],
  )(agg1, x, W1, b1.reshape(1, h1_dim), W2)

  agg2 = seg2(p, src, dst, z2)

  z = pl.pallas_call(
      _layer2_body,
      grid=(1,),
      in_specs=[
          pl.BlockSpec((NC, n, h2), lambda i: (0, 0, 0)),
          pl.BlockSpec((n, h2), lambda i: (0, 0)),
          pl.BlockSpec((n, h2), lambda i: (0, 0)),
          pl.BlockSpec((1, h2), lambda i: (0, 0)),
      ],
      out_specs=pl.BlockSpec((n, h2), lambda i: (0, 0)),
      out_shape=jax.ShapeDtypeStruct((n, h2), jnp.float32),
  )(agg2, p, invd, b2.reshape(1, h2))

  ablk = 1024
  grid_a = pl.cdiv(n, ablk)
  adj = pl.pallas_call(
      _decoder_body,
      grid=(grid_a, grid_a),
      in_specs=[
          pl.BlockSpec((ablk, h2), lambda i, j: (i, 0)),
          pl.BlockSpec((ablk, h2), lambda i, j: (j, 0)),
      ],
      out_specs=pl.BlockSpec((ablk, ablk), lambda i, j: (i, j)),
      out_shape=jax.ShapeDtypeStruct((n, n), jnp.float32),
  )(z, z)

  return (z, adj)


# speedup vs baseline: 9.1717x; 1.7904x over previous
import functools

import jax
import jax.numpy as jnp
from jax import lax
from jax.experimental import pallas as pl
from jax.experimental.pallas import tpu as pltpu
from jax.experimental.pallas import tpu_sc as plsc

NC = 2
NS = 16
NW = NC * NS


def _make_seg_sum(n, e, d, ch, nbuf):
  et = e // NW
  nchunks = et // ch
  nslot = 2 * nbuf
  assert et % ch == 0 and nchunks % nslot == 0
  rpt = (n // NS + 7) // 8 * 8
  npad = rpt * NS
  mesh = plsc.VectorSubcoreMesh(core_axis_name="c", subcore_axis_name="s")

  @functools.partial(
      pl.kernel,
      mesh=mesh,
      compiler_params=pltpu.CompilerParams(use_tc_tiling_on_sc=False),
      out_type=jax.ShapeDtypeStruct((NC, npad, d), jnp.float32),
      scratch_types=[
          pltpu.VMEM((nslot, 2, ch), jnp.int32),
          pltpu.VMEM((nbuf, ch, d), jnp.float32),
          pltpu.VMEM_SHARED((npad, d), jnp.float32),
          pltpu.SemaphoreType.DMA((nslot,)),
          pltpu.SemaphoreType.DMA((nbuf,)),
      ],
  )
  def seg(tab_hbm, eidx_hbm, zrows_hbm, out_hbm,
          idx_v, rows_v, table_s, isems, gsems):
    c = lax.axis_index("c")
    s = lax.axis_index("s")
    wid = s * NC + c
    cbase = wid * nchunks

    pltpu.sync_copy(zrows_hbm, table_s.at[pl.ds(s * rpt, rpt)])

    for q in range(nslot):
      pltpu.async_copy(eidx_hbm.at[cbase + q], idx_v.at[q], isems.at[q])
    for b in range(nbuf):
      pltpu.make_async_copy(eidx_hbm.at[0], idx_v.at[b], isems.at[b]).wait()
      pltpu.async_copy(tab_hbm.at[idx_v.at[b, 0]], rows_v.at[b], gsems.at[b])

    plsc.subcore_barrier()

    def group(g, carry):
      for u in range(nslot):
        j = g * nslot + u
        b = u % nbuf
        pltpu.make_async_copy(tab_hbm.at[pl.ds(0, ch)], rows_v.at[b],
                              gsems.at[b]).wait()
        pltpu.sync_copy(rows_v.at[b], table_s.at[idx_v.at[u, 1]], add=True)

        @pl.when(j + nslot < nchunks)
        def _():
          pltpu.async_copy(eidx_hbm.at[cbase + j + nslot], idx_v.at[u],
                           isems.at[u])

        qn = (u + nbuf) % nslot
        @pl.when(j + nbuf < nchunks)
        def _():
          pltpu.make_async_copy(eidx_hbm.at[0], idx_v.at[qn],
                                isems.at[qn]).wait()
          pltpu.async_copy(tab_hbm.at[idx_v.at[qn, 0]], rows_v.at[b],
                           gsems.at[b])
      return carry

    lax.fori_loop(0, nchunks // nslot, group, 0, unroll=False)
    plsc.subcore_barrier()

    pltpu.sync_copy(table_s.at[pl.ds(s * rpt, rpt)],
                    out_hbm.at[c, pl.ds(s * rpt, rpt)])

  return seg


def _layer1_body(agg_ref, x_ref, w1_ref, b1_ref, w2_ref, p_ref, invd_ref):
  agg = agg_ref[0] + agg_ref[1]
  din = x_ref.shape[1]
  aggx = agg[:, :din] + x_ref[...]
  deg = jnp.sum(agg[:, din:din + 16], axis=1, keepdims=True)
  inv = 1.0 / (deg + 1.0)
  h = aggx * inv
  h1 = jnp.maximum(
      jnp.dot(h, w1_ref[...], preferred_element_type=jnp.float32)
      + b1_ref[...], 0.0)
  p_ref[...] = jnp.dot(h1, w2_ref[...], preferred_element_type=jnp.float32)
  invd_ref[...] = jnp.broadcast_to(inv, invd_ref.shape)


def _layer2_body(agg_ref, p_ref, invd_ref, b2_ref, z_ref):
  z_ref[...] = ((agg_ref[0] + agg_ref[1] + p_ref[...]) * invd_ref[...]
                + b2_ref[...])


def _decoder_body(zr_ref, zc_ref, out_ref):
  out_ref[...] = lax.dot_general(
      zr_ref[...], zc_ref[...], (((1,), (1,)), ((), ())),
      preferred_element_type=jnp.float32)


def kernel(x, edge_index, W1, b1, W2, b2):
  n, din = x.shape
  e = edge_index.shape[1]
  h1_dim = W1.shape[1]
  h2 = W2.shape[1]
  dp = din + 16

  src = edge_index[0]
  dst = edge_index[1]

  xa = jnp.concatenate(
      [x, jnp.ones((n, 1), jnp.float32), jnp.zeros((n, 15), jnp.float32)],
      axis=1)

  rpt = (n // NS + 7) // 8 * 8
  z1 = jnp.zeros((rpt, dp), jnp.float32)
  z2 = jnp.zeros((rpt, h2), jnp.float32)

  ch1, ch2 = 50, 100
  eidx1 = jnp.stack([src.reshape(-1, ch1), dst.reshape(-1, ch1)], axis=1)
  eidx2 = jnp.stack([src.reshape(-1, ch2), dst.reshape(-1, ch2)], axis=1)
  seg1 = _make_seg_sum(n, e, dp, ch1, nbuf=5)
  seg2 = _make_seg_sum(n, e, h2, ch2, nbuf=5)

  agg1 = seg1(xa, eidx1, z1)

  rblk = 2000
  grid1 = n // rblk
  p, invd = pl.pallas_call(
      _layer1_body,
      grid=(grid1,),
      in_specs=[
          pl.BlockSpec((NC, rblk, dp), lambda i: (0, i, 0)),
          pl.BlockSpec((rblk, din), lambda i: (i, 0)),
          pl.BlockSpec((din, h1_dim), lambda i: (0, 0)),
          pl.BlockSpec((1, h1_dim), lambda i: (0, 0)),
          pl.BlockSpec((h1_dim, h2), lambda i: (0, 0)),
      ],
      out_specs=[
          pl.BlockSpec((rblk, h2), lambda i: (i, 0)),
          pl.BlockSpec((rblk, h2), lambda i: (i, 0)),
      ],
      out_shape=[
          jax.ShapeDtypeStruct((n, h2), jnp.float32),
          jax.ShapeDtypeStruct((n, h2), jnp.float32),
      ],
  )(agg1, x, W1, b1.reshape(1, h1_dim), W2)

  agg2 = seg2(p, eidx2, z2)

  z = pl.pallas_call(
      _layer2_body,
      grid=(1,),
      in_specs=[
          pl.BlockSpec((NC, n, h2), lambda i: (0, 0, 0)),
          pl.BlockSpec((n, h2), lambda i: (0, 0)),
          pl.BlockSpec((n, h2), lambda i: (0, 0)),
          pl.BlockSpec((1, h2), lambda i: (0, 0)),
      ],
      out_specs=pl.BlockSpec((n, h2), lambda i: (0, 0)),
      out_shape=jax.ShapeDtypeStruct((n, h2), jnp.float32),
  )(agg2, p, invd, b2.reshape(1, h2))

  ablk = 1024
  grid_a = pl.cdiv(n, ablk)
  adj = pl.pallas_call(
      _decoder_body,
      grid=(grid_a, grid_a),
      in_specs=[
          pl.BlockSpec((ablk, h2), lambda i, j: (i, 0)),
          pl.BlockSpec((ablk, h2), lambda i, j: (j, 0)),
      ],
      out_specs=pl.BlockSpec((ablk, ablk), lambda i, j: (i, j)),
      out_shape=jax.ShapeDtypeStruct((n, n), jnp.float32),
  )(z, z)

  return (z, adj)


# speedup vs baseline: 10.1716x; 1.1090x over previous
import functools

import jax
import jax.numpy as jnp
from jax import lax
from jax.experimental import pallas as pl
from jax.experimental.pallas import tpu as pltpu
from jax.experimental.pallas import tpu_sc as plsc

NC = 2
NS = 16
NW = NC * NS
DG = 16


def _make_seg_sum(n, e, d, ch, nbuf, with_deg):
  et = e // NW
  nchunks = et // ch
  nslot = 2 * nbuf
  assert et % ch == 0 and nchunks % nslot == 0 and ch % 8 == 0
  rpt = (n // NS + 7) // 8 * 8
  npad = rpt * NS

  out_type = [jax.ShapeDtypeStruct((NC, npad, d), jnp.float32)]
  scratch = [
      pltpu.VMEM((nslot, 2, ch), jnp.int32),
      pltpu.VMEM((nbuf, ch, d), jnp.float32),
      pltpu.VMEM_SHARED((npad, d), jnp.float32),
      pltpu.SemaphoreType.DMA((nslot,)),
      pltpu.SemaphoreType.DMA((nslot,)),
      pltpu.SemaphoreType.DMA((nbuf,)),
  ]
  if with_deg:
    out_type.append(jax.ShapeDtypeStruct((NC, npad, DG), jnp.float32))
    scratch.append(pltpu.VMEM((ch, DG), jnp.float32))
    scratch.append(pltpu.VMEM_SHARED((npad, DG), jnp.float32))

  mesh = plsc.VectorSubcoreMesh(core_axis_name="c", subcore_axis_name="s")

  def body(tab_hbm, src_hbm, dst_hbm, zrows_hbm, zdeg_hbm, out_hbm, deg_hbm,
           idx_v, rows_v, table_s, isems_s, isems_d, gsems,
           ones_v, degtab_s):
    c = lax.axis_index("c")
    s = lax.axis_index("s")
    wid = s * NC + c
    ebase = wid * et

    pltpu.sync_copy(zrows_hbm, table_s.at[pl.ds(s * rpt, rpt)])
    if with_deg:
      pltpu.sync_copy(zdeg_hbm, degtab_s.at[pl.ds(s * rpt, rpt)])
      for i in range(ch):
        ones_v[i] = jnp.ones((DG,), jnp.float32)

    for q in range(nslot):
      eb = ebase + q * ch
      pltpu.async_copy(src_hbm.at[pl.ds(eb, ch)], idx_v.at[q, 0],
                       isems_s.at[q])
      pltpu.async_copy(dst_hbm.at[pl.ds(eb, ch)], idx_v.at[q, 1],
                       isems_d.at[q])
    for b in range(nbuf):
      pltpu.make_async_copy(src_hbm.at[pl.ds(0, ch)], idx_v.at[b, 0],
                            isems_s.at[b]).wait()
      pltpu.async_copy(tab_hbm.at[idx_v.at[b, 0]], rows_v.at[b], gsems.at[b])

    plsc.subcore_barrier()

    def group(g, carry):
      for u in range(nslot):
        j = g * nslot + u
        b = u % nbuf
        pltpu.make_async_copy(tab_hbm.at[pl.ds(0, ch)], rows_v.at[b],
                              gsems.at[b]).wait()
        pltpu.make_async_copy(dst_hbm.at[pl.ds(0, ch)], idx_v.at[u, 1],
                              isems_d.at[u]).wait()
        pltpu.sync_copy(rows_v.at[b], table_s.at[idx_v.at[u, 1]], add=True)
        if with_deg:
          pltpu.sync_copy(ones_v, degtab_s.at[idx_v.at[u, 1]], add=True)

        @pl.when(j + nslot < nchunks)
        def _():
          eb = ebase + j * ch + nslot * ch
          pltpu.async_copy(src_hbm.at[pl.ds(eb, ch)], idx_v.at[u, 0],
                           isems_s.at[u])
          pltpu.async_copy(dst_hbm.at[pl.ds(eb, ch)], idx_v.at[u, 1],
                           isems_d.at[u])

        qn = (u + nbuf) % nslot
        @pl.when(j + nbuf < nchunks)
        def _():
          pltpu.make_async_copy(src_hbm.at[pl.ds(0, ch)], idx_v.at[qn, 0],
                                isems_s.at[qn]).wait()
          pltpu.async_copy(tab_hbm.at[idx_v.at[qn, 0]], rows_v.at[b],
                           gsems.at[b])
      return carry

    lax.fori_loop(0, nchunks // nslot, group, 0, unroll=False)
    plsc.subcore_barrier()

    pltpu.sync_copy(table_s.at[pl.ds(s * rpt, rpt)],
                    out_hbm.at[c, pl.ds(s * rpt, rpt)])
    if with_deg:
      pltpu.sync_copy(degtab_s.at[pl.ds(s * rpt, rpt)],
                      deg_hbm.at[c, pl.ds(s * rpt, rpt)])

  kern = functools.partial(
      pl.kernel,
      mesh=mesh,
      compiler_params=pltpu.CompilerParams(use_tc_tiling_on_sc=False),
      out_type=tuple(out_type) if with_deg else out_type[0],
      scratch_types=scratch,
  )

  if with_deg:
    @kern
    def seg(tab, src, dst, zrows, zdeg, out, deg,
            idx_v, rows_v, table_s, isems_s, isems_d, gsems, ones_v, degtab_s):
      body(tab, src, dst, zrows, zdeg, out, deg,
           idx_v, rows_v, table_s, isems_s, isems_d, gsems, ones_v, degtab_s)
  else:
    @kern
    def seg(tab, src, dst, zrows, out,
            idx_v, rows_v, table_s, isems_s, isems_d, gsems):
      body(tab, src, dst, zrows, None, out, None,
           idx_v, rows_v, table_s, isems_s, isems_d, gsems, None, None)

  return seg


def _layer1_body(feat_ref, degt_ref, x_ref, w1_ref, b1_ref, w2_ref,
                 p_ref, invd_ref):
  aggx = feat_ref[0] + feat_ref[1] + x_ref[...]
  deg16 = degt_ref[0] + degt_ref[1]
  inv = 1.0 / (jnp.max(deg16, axis=1, keepdims=True) + 1.0)
  h = aggx * inv
  h1 = jnp.maximum(
      jnp.dot(h, w1_ref[...], preferred_element_type=jnp.float32)
      + b1_ref[...], 0.0)
  p_ref[...] = jnp.dot(h1, w2_ref[...], preferred_element_type=jnp.float32)
  invd_ref[...] = jnp.broadcast_to(inv, invd_ref.shape)


def _layer2_body(agg_ref, p_ref, invd_ref, b2_ref, z_ref):
  z_ref[...] = ((agg_ref[0] + agg_ref[1] + p_ref[...]) * invd_ref[...]
                + b2_ref[...])


def _decoder_body(zr_ref, zc_ref, out_ref):
  out_ref[...] = lax.dot_general(
      zr_ref[...], zc_ref[...], (((1,), (1,)), ((), ())),
      preferred_element_type=jnp.float32)


def kernel(x, edge_index, W1, b1, W2, b2):
  n, din = x.shape
  e = edge_index.shape[1]
  h1_dim = W1.shape[1]
  h2 = W2.shape[1]

  src = edge_index[0]
  dst = edge_index[1]

  rpt = (n // NS + 7) // 8 * 8
  z1 = jnp.zeros((rpt, din), jnp.float32)
  zd = jnp.zeros((rpt, DG), jnp.float32)
  z2 = jnp.zeros((rpt, h2), jnp.float32)

  seg1 = _make_seg_sum(n, e, din, 40, 5, with_deg=True)
  seg2 = _make_seg_sum(n, e, h2, 40, 5, with_deg=False)

  feat, degt = seg1(x, src, dst, z1, zd)

  rblk = 2000
  grid1 = n // rblk
  p, invd = pl.pallas_call(
      _layer1_body,
      grid=(grid1,),
      in_specs=[
          pl.BlockSpec((NC, rblk, din), lambda i: (0, i, 0)),
          pl.BlockSpec((NC, rblk, DG), lambda i: (0, i, 0)),
          pl.BlockSpec((rblk, din), lambda i: (i, 0)),
          pl.BlockSpec((din, h1_dim), lambda i: (0, 0)),
          pl.BlockSpec((1, h1_dim), lambda i: (0, 0)),
          pl.BlockSpec((h1_dim, h2), lambda i: (0, 0)),
      ],
      out_specs=[
          pl.BlockSpec((rblk, h2), lambda i: (i, 0)),
          pl.BlockSpec((rblk, h2), lambda i: (i, 0)),
      ],
      out_shape=[
          jax.ShapeDtypeStruct((n, h2), jnp.float32),
          jax.ShapeDtypeStruct((n, h2), jnp.float32),
      ],
  )(feat, degt, x, W1, b1.reshape(1, h1_dim), W2)

  agg2 = seg2(p, src, dst, z2)

  z = pl.pallas_call(
      _layer2_body,
      grid=(grid1,),
      in_specs=[
          pl.BlockSpec((NC, rblk, h2), lambda i: (0, i, 0)),
          pl.BlockSpec((rblk, h2), lambda i: (i, 0)),
          pl.BlockSpec((rblk, h2), lambda i: (i, 0)),
          pl.BlockSpec((1, h2), lambda i: (0, 0)),
      ],
      out_specs=pl.BlockSpec((rblk, h2), lambda i: (i, 0)),
      out_shape=jax.ShapeDtypeStruct((n, h2), jnp.float32),
  )(agg2, p, invd, b2.reshape(1, h2))

  ablk = 1024
  grid_a = pl.cdiv(n, ablk)
  adj = pl.pallas_call(
      _decoder_body,
      grid=(grid_a, grid_a),
      in_specs=[
          pl.BlockSpec((ablk, h2), lambda i, j: (i, 0)),
          pl.BlockSpec((ablk, h2), lambda i, j: (j, 0)),
      ],
      out_specs=pl.BlockSpec((ablk, ablk), lambda i, j: (i, j)),
      out_shape=jax.ShapeDtypeStruct((n, n), jnp.float32),
  )(z, z)

  return (z, adj)


# speedup vs baseline: 10.9024x; 1.0718x over previous
import functools

import jax
import jax.numpy as jnp
from jax import lax
from jax.experimental import pallas as pl
from jax.experimental.pallas import tpu as pltpu
from jax.experimental.pallas import tpu_sc as plsc

NC = 2
NS = 16
NW = NC * NS
DG = 16


def _make_seg_sum(n, e, d, ch, nbuf, with_deg):
  et = e // NW
  nchunks = et // ch
  nslot = 2 * nbuf
  ngroups = nchunks // nslot
  ntail = nchunks % nslot
  assert et % ch == 0 and ch % 8 == 0 and nchunks >= nslot
  rpt = (n // NS + 7) // 8 * 8
  npad = rpt * NS

  out_type = [jax.ShapeDtypeStruct((NC, npad, d), jnp.float32)]
  scratch = [
      pltpu.VMEM((nslot, 2, ch), jnp.int32),
      pltpu.VMEM((nbuf, ch, d), jnp.float32),
      pltpu.VMEM_SHARED((npad, d), jnp.float32),
      pltpu.SemaphoreType.DMA((nslot,)),
      pltpu.SemaphoreType.DMA((nslot,)),
      pltpu.SemaphoreType.DMA((nbuf,)),
  ]
  if with_deg:
    out_type.append(jax.ShapeDtypeStruct((NC, npad, DG), jnp.float32))
    scratch.append(pltpu.VMEM((ch, DG), jnp.float32))
    scratch.append(pltpu.VMEM_SHARED((npad, DG), jnp.float32))

  mesh = plsc.VectorSubcoreMesh(core_axis_name="c", subcore_axis_name="s")

  def body(tab_hbm, ei_hbm, zrows_hbm, zdeg_hbm, out_hbm, deg_hbm,
           idx_v, rows_v, table_s, isems_s, isems_d, gsems,
           ones_v, degtab_s):
    c = lax.axis_index("c")
    s = lax.axis_index("s")
    wid = s * NC + c
    ebase = wid * et

    pltpu.sync_copy(zrows_hbm, table_s.at[pl.ds(s * rpt, rpt)])
    if with_deg:
      pltpu.sync_copy(zdeg_hbm, degtab_s.at[pl.ds(s * rpt, rpt)])
      for i in range(ch):
        ones_v[i] = jnp.ones((DG,), jnp.float32)

    def load_idx(j, q):
      eb = ebase + j * ch
      pltpu.async_copy(ei_hbm.at[0, pl.ds(eb, ch)], idx_v.at[q, 0],
                       isems_s.at[q])
      pltpu.async_copy(ei_hbm.at[1, pl.ds(eb, ch)], idx_v.at[q, 1],
                       isems_d.at[q])

    def issue_gather(q, b):
      pltpu.make_async_copy(ei_hbm.at[0, pl.ds(0, ch)], idx_v.at[q, 0],
                            isems_s.at[q]).wait()
      pltpu.async_copy(tab_hbm.at[idx_v.at[q, 0]], rows_v.at[b], gsems.at[b])

    for q in range(nslot):
      load_idx(q, q)
    for b in range(nbuf):
      issue_gather(b, b)

    plsc.subcore_barrier()

    def stage(j, u, guard):
      b = u % nbuf
      pltpu.make_async_copy(tab_hbm.at[pl.ds(0, ch)], rows_v.at[b],
                            gsems.at[b]).wait()
      pltpu.make_async_copy(ei_hbm.at[0, pl.ds(0, ch)], idx_v.at[u, 1],
                            isems_d.at[u]).wait()
      pltpu.sync_copy(rows_v.at[b], table_s.at[idx_v.at[u, 1]], add=True)
      if with_deg:
        pltpu.sync_copy(ones_v, degtab_s.at[idx_v.at[u, 1]], add=True)

      guard(j + nslot < nchunks, lambda: load_idx(j + nslot, u))
      qn = (u + nbuf) % nslot
      guard(j + nbuf < nchunks, lambda: issue_gather(qn, b))

    def traced_guard(cond, fn):
      pl.when(cond)(fn)

    def static_guard(cond, fn):
      if cond:
        fn()

    def group(g, carry):
      for u in range(nslot):
        stage(g * nslot + u, u, traced_guard)
      return carry

    lax.fori_loop(0, ngroups, group, 0, unroll=False)
    for u in range(ntail):
      stage(ngroups * nslot + u, u, static_guard)
    plsc.subcore_barrier()

    pltpu.sync_copy(table_s.at[pl.ds(s * rpt, rpt)],
                    out_hbm.at[c, pl.ds(s * rpt, rpt)])
    if with_deg:
      pltpu.sync_copy(degtab_s.at[pl.ds(s * rpt, rpt)],
                      deg_hbm.at[c, pl.ds(s * rpt, rpt)])

  kern = functools.partial(
      pl.kernel,
      mesh=mesh,
      compiler_params=pltpu.CompilerParams(use_tc_tiling_on_sc=False),
      out_type=tuple(out_type) if with_deg else out_type[0],
      scratch_types=scratch,
  )

  if with_deg:
    @kern
    def seg(tab, ei, zrows, zdeg, out, deg,
            idx_v, rows_v, table_s, isems_s, isems_d, gsems, ones_v, degtab_s):
      body(tab, ei, zrows, zdeg, out, deg,
           idx_v, rows_v, table_s, isems_s, isems_d, gsems, ones_v, degtab_s)
  else:
    @kern
    def seg(tab, ei, zrows, out,
            idx_v, rows_v, table_s, isems_s, isems_d, gsems):
      body(tab, ei, zrows, None, out, None,
           idx_v, rows_v, table_s, isems_s, isems_d, gsems, None, None)

  return seg


def _layer1_body(feat_ref, degt_ref, x_ref, w1_ref, b1_ref, w2_ref,
                 p_ref, invd_ref):
  aggx = feat_ref[0] + feat_ref[1] + x_ref[...]
  deg16 = degt_ref[0] + degt_ref[1]
  inv = 1.0 / (jnp.max(deg16, axis=1, keepdims=True) + 1.0)
  h = aggx * inv
  h1 = jnp.maximum(
      jnp.dot(h, w1_ref[...], preferred_element_type=jnp.float32)
      + b1_ref[...], 0.0)
  p_ref[...] = jnp.dot(h1, w2_ref[...], preferred_element_type=jnp.float32)
  invd_ref[...] = jnp.broadcast_to(inv, invd_ref.shape)


def _layer2_body(agg_ref, p_ref, invd_ref, b2_ref, z_ref):
  z_ref[...] = ((agg_ref[0] + agg_ref[1] + p_ref[...]) * invd_ref[...]
                + b2_ref[...])


def _decoder_body(zr_ref, zc_ref, out_ref):
  out_ref[...] = lax.dot_general(
      zr_ref[...], zc_ref[...], (((1,), (1,)), ((), ())),
      preferred_element_type=jnp.float32)


def kernel(x, edge_index, W1, b1, W2, b2):
  n, din = x.shape
  e = edge_index.shape[1]
  h1_dim = W1.shape[1]
  h2 = W2.shape[1]

  rpt = (n // NS + 7) // 8 * 8
  z1 = jnp.zeros((rpt, din), jnp.float32)
  zd = jnp.zeros((rpt, DG), jnp.float32)
  z2 = jnp.zeros((rpt, h2), jnp.float32)

  seg1 = _make_seg_sum(n, e, din, 80, 3, with_deg=True)
  seg2 = _make_seg_sum(n, e, h2, 80, 3, with_deg=False)

  feat, degt = seg1(x, edge_index, z1, zd)

  rblk = 2000
  grid1 = n // rblk
  p, invd = pl.pallas_call(
      _layer1_body,
      grid=(grid1,),
      in_specs=[
          pl.BlockSpec((NC, rblk, din), lambda i: (0, i, 0)),
          pl.BlockSpec((NC, rblk, DG), lambda i: (0, i, 0)),
          pl.BlockSpec((rblk, din), lambda i: (i, 0)),
          pl.BlockSpec((din, h1_dim), lambda i: (0, 0)),
          pl.BlockSpec((1, h1_dim), lambda i: (0, 0)),
          pl.BlockSpec((h1_dim, h2), lambda i: (0, 0)),
      ],
      out_specs=[
          pl.BlockSpec((rblk, h2), lambda i: (i, 0)),
          pl.BlockSpec((rblk, h2), lambda i: (i, 0)),
      ],
      out_shape=[
          jax.ShapeDtypeStruct((n, h2), jnp.float32),
          jax.ShapeDtypeStruct((n, h2), jnp.float32),
      ],
  )(feat, degt, x, W1, b1.reshape(1, h1_dim), W2)

  agg2 = seg2(p, edge_index, z2)

  z = pl.pallas_call(
      _layer2_body,
      grid=(grid1,),
      in_specs=[
          pl.BlockSpec((NC, rblk, h2), lambda i: (0, i, 0)),
          pl.BlockSpec((rblk, h2), lambda i: (i, 0)),
          pl.BlockSpec((rblk, h2), lambda i: (i, 0)),
          pl.BlockSpec((1, h2), lambda i: (0, 0)),
      ],
      out_specs=pl.BlockSpec((rblk, h2), lambda i: (i, 0)),
      out_shape=jax.ShapeDtypeStruct((n, h2), jnp.float32),
  )(agg2, p, invd, b2.reshape(1, h2))

  ablk = 1024
  grid_a = pl.cdiv(n, ablk)
  adj = pl.pallas_call(
      _decoder_body,
      grid=(grid_a, grid_a),
      in_specs=[
          pl.BlockSpec((ablk, h2), lambda i, j: (i, 0)),
          pl.BlockSpec((ablk, h2), lambda i, j: (j, 0)),
      ],
      out_specs=pl.BlockSpec((ablk, ablk), lambda i, j: (i, j)),
      out_shape=jax.ShapeDtypeStruct((n, n), jnp.float32),
  )(z, z)

  return (z, adj)
